# baseline (speedup 1.0000x reference)
import jax
import jax.numpy as jnp
from jax.experimental import pallas as pl

N_U = 100000
M_I = 100000
DIM = 64
LAYERS = 3


def _dot_body(u_ref, i_ref, s_ref, o_ref):
    o_ref[:] = jnp.sum(u_ref[:] * i_ref[:], axis=1) + s_ref[0] * 0.0


def kernel(users, items, user_emb, item_emb, edge_u, edge_i, graph_vals):
    N = N_U + M_I
    eu = edge_u.astype(jnp.int32)
    ei = edge_i.astype(jnp.int32)
    row = jnp.concatenate([eu, ei + N_U], axis=0)
    col = jnp.concatenate([ei + N_U, eu], axis=0)
    row_s, col_s = jax.lax.sort([row, col], num_keys=1)
    row_ptr = jnp.searchsorted(row_s, jnp.arange(N + 1, dtype=jnp.int32))
    deg = jnp.diff(row_ptr).astype(jnp.float32)
    dmax = jnp.maximum(deg, 1.0)
    dinv = 1.0 / dmax

    vals = jnp.concatenate([graph_vals, graph_vals], axis=0)
    all_emb = jnp.concatenate([user_emb, item_emb], axis=0)
    acc = all_emb
    x = all_emb
    for _ in range(LAYERS):
        gathered = x[col] * vals[:, None]
        x = jnp.zeros((N, DIM), dtype=x.dtype).at[row].add(gathered)
        acc = acc + x
    out = acc * 0.25
    u = out[users]
    i = out[items + N_U]
    s = (col_s[:1].astype(jnp.float32) + dinv[:1])
    return pl.pallas_call(
        _dot_body,
        out_shape=jax.ShapeDtypeStruct((u.shape[0],), jnp.float32),
    )(u, i, s)


# speedup vs baseline: 5.2942x; 5.2942x over previous
import functools

import jax
import jax.numpy as jnp
from jax import lax
from jax.experimental import pallas as pl
from jax.experimental.pallas import tpu as pltpu
from jax.experimental.pallas import tpu_sc as plsc

N_U = 100000
M_I = 100000
SIDE = 100000
SIDE_P = 102400
NP = 2 * SIDE_P
D = 64
E = 600000
NC = 2
NS = 16
CK = 128
NCH = 294
EP_SIDE = NCH * CK * NS
W = 20480
PW = 5
RT = W // NS
WD = SIDE_P
RTD = WD // NS

_f32 = jnp.float32
_i32 = jnp.int32


def _mesh():
    return plsc.VectorSubcoreMesh(core_axis_name="c", subcore_axis_name="s")


def _clamp_to_window(dstb, dstl, base, w):
    for v in range(CK // 16):
        sl = pl.ds(v * 16, 16)
        d = dstb[sl]
        loc = d - base
        ok = (loc >= 0) & (loc < w)
        dstl[sl] = jnp.where(ok, loc, w)


def _deg_body(dst_hbm, deg_hbm, acc, dstb, dstl, onesb, degb):
    c = lax.axis_index("c")
    s = lax.axis_index("s")

    zf = jnp.zeros((16,), _f32)

    @pl.loop(0, CK)
    def _fill_ones(r):
        onesb[r, pl.ds(0, 16)] = jnp.full((16,), 1.0, _f32)

    @pl.loop(0, 64)
    def _zb(r):
        degb[r, pl.ds(0, 16)] = zf

    @pl.loop(0, RTD // 64)
    def _zero(i):
        pltpu.sync_copy(degb, acc.at[pl.ds(s * RTD + i * 64, 64)])

    plsc.subcore_barrier()

    @pl.loop(0, NCH)
    def _chunks(j):
        pltpu.sync_copy(dst_hbm.at[c, s, j], dstb)
        _clamp_to_window(dstb, dstl, 0, WD)
        pltpu.sync_copy(onesb, acc.at[dstl], add=True)

    plsc.subcore_barrier()

    pltpu.sync_copy(acc.at[pl.ds(s * RTD, RTD)],
                    deg_hbm.at[pl.ds(c * SIDE_P + s * RTD, RTD)])


def _deg_call(dst2):
    kern = pl.kernel(
        _deg_body,
        out_type=jax.ShapeDtypeStruct((NP, 16), _f32),
        mesh=_mesh(),
        compiler_params=pltpu.CompilerParams(use_tc_tiling_on_sc=False),
        scratch_types=[
            pltpu.VMEM_SHARED((WD + 8, 16), _f32),
            pltpu.VMEM((CK,), _i32),
            pltpu.VMEM((CK,), _i32),
            pltpu.VMEM((CK, 16), _f32),
            pltpu.VMEM((64, 16), _f32),
        ],
    )
    return kern(dst2)


def _scale_body(x_ref, d_ref, o_ref):
    o_ref[:, :] = x_ref[:, :] * d_ref[:, :]


def _u0_call(all_emb_p, dsqi2d):
    return pl.pallas_call(
        _scale_body,
        grid=(NP // 1024,),
        in_specs=[
            pl.BlockSpec((1024, D), lambda i: (i, 0)),
            pl.BlockSpec((1024, 1), lambda i: (i, 0)),
        ],
        out_specs=pl.BlockSpec((1024, D), lambda i: (i, 0)),
        out_shape=jax.ShapeDtypeStruct((NP, D), _f32),
    )(all_emb_p, dsqi2d)


def _layer_body(uprev, src, dst, dinv, unext,
                acc,
                colb0, colb1, dstb0, dstb1, dstl0, dstl1,
                gb0, gb1, wb, dinvb,
                gs0, gs1, ss0, ss1):
    c = lax.axis_index("c")
    s = lax.axis_index("s")
    bufs = ((colb0, dstb0, dstl0, gb0, gs0, ss0),
            (colb1, dstb1, dstl1, gb1, gs1, ss1))

    zf = jnp.zeros((16,), _f32)

    for b in range(2):
        colb, dstb, _, gb, gs, _ = bufs[b]
        pltpu.sync_copy(src.at[c, s, b], colb)
        pltpu.sync_copy(dst.at[c, s, b], dstb)
        pltpu.async_copy(uprev.at[colb], gb, gs)

    for p in range(PW):
        base = p * W

        @pl.loop(0, 64)
        def _zwb(r):
            for cc in range(D // 16):
                wb[r, pl.ds(cc * 16, 16)] = zf

        @pl.loop(0, RT // 64)
        def _zero(i):
            pltpu.sync_copy(wb, acc.at[pl.ds(s * RT + i * 64, 64)])

        plsc.subcore_barrier()

        @pl.loop(0, NCH // 2)
        def _chunks(i):
            for b in range(2):
                colb, dstb, dstl, gb, gs, ss = bufs[b]
                j = i * 2 + b
                _clamp_to_window(dstb, dstl, base, W)
                pltpu.make_async_copy(uprev.at[colb], gb, gs).wait()
                pltpu.async_copy(gb, acc.at[dstl], ss, add=True).wait()
                jn = lax.rem(j + 2, NCH)
                pltpu.sync_copy(src.at[c, s, jn], colb)
                pltpu.sync_copy(dst.at[c, s, jn], dstb)
                pltpu.async_copy(uprev.at[colb], gb, gs)

        plsc.subcore_barrier()

        g0 = c * SIDE_P + base + s * RT
        pltpu.sync_copy(dinv.at[pl.ds(g0, RT)], dinvb)

        @pl.loop(0, RT // 64)
        def _wb(t):
            pltpu.sync_copy(acc.at[pl.ds(s * RT + t * 64, 64)], wb)

            @pl.loop(0, 4)
            def _rows(g):
                dvv = dinvb[pl.ds(t * 64 + g * 16, 16)]
                for k in range(16):
                    r = g * 16 + k
                    dv = dvv[k]
                    for cc in range(D // 16):
                        sl = pl.ds(cc * 16, 16)
                        wb[r, sl] = wb[r, sl] * dv

            pltpu.sync_copy(wb, unext.at[pl.ds(g0 + t * 64, 64)])

    for b in range(2):
        colb, _, _, gb, gs, _ = bufs[b]
        pltpu.make_async_copy(uprev.at[colb], gb, gs).wait()


def _layer_call(uprev, src2, dst2, dinv):
    kern = pl.kernel(
        _layer_body,
        out_type=jax.ShapeDtypeStruct((NP, D), _f32),
        mesh=_mesh(),
        compiler_params=pltpu.CompilerParams(use_tc_tiling_on_sc=False),
        scratch_types=[
            pltpu.VMEM_SHARED((W + 8, D), _f32),
            pltpu.VMEM((CK,), _i32),
            pltpu.VMEM((CK,), _i32),
            pltpu.VMEM((CK,), _i32),
            pltpu.VMEM((CK,), _i32),
            pltpu.VMEM((CK,), _i32),
            pltpu.VMEM((CK,), _i32),
            pltpu.VMEM((CK, D), _f32),
            pltpu.VMEM((CK, D), _f32),
            pltpu.VMEM((64, D), _f32),
            pltpu.VMEM((RT,), _f32),
            pltpu.SemaphoreType.DMA,
            pltpu.SemaphoreType.DMA,
            pltpu.SemaphoreType.DMA,
            pltpu.SemaphoreType.DMA,
        ],
    )
    return kern(uprev, src2, dst2, dinv)


def _final_body(u0, u1, u2, u3, dsq16, uq, iq, out,
                uqb, iqb, gu0, gu1, gu2, gu3, gi0, gi1, gi2, gi3,
                du16, di16, ob16):
    c = lax.axis_index("c")
    s = lax.axis_index("s")
    wid = s * NC + c
    bq = 4096 // (NC * NS)
    sb = 64

    @pl.loop(0, bq // sb)
    def _sub(sr):
        base = wid * bq + sr * sb
        pltpu.sync_copy(uq.at[pl.ds(base, sb)], uqb)
        pltpu.sync_copy(iq.at[pl.ds(base, sb)], iqb)
        for tab, gb in ((u0, gu0), (u1, gu1), (u2, gu2), (u3, gu3)):
            pltpu.sync_copy(tab.at[uqb], gb)
        for tab, gb in ((u0, gi0), (u1, gi1), (u2, gi2), (u3, gi3)):
            pltpu.sync_copy(tab.at[iqb], gb)
        pltpu.sync_copy(dsq16.at[uqb], du16)
        pltpu.sync_copy(dsq16.at[iqb], di16)

        @pl.loop(0, sb)
        def _rows(r):
            tot = jnp.zeros((16,), _f32)
            for cc in range(D // 16):
                sl = pl.ds(cc * 16, 16)
                su = gu0[r, sl] + gu1[r, sl] + gu2[r, sl] + gu3[r, sl]
                si = gi0[r, sl] + gi1[r, sl] + gi2[r, sl] + gi3[r, sl]
                tot = tot + su * si
            ob16[r, pl.ds(0, 16)] = (tot * du16[r, pl.ds(0, 16)]
                                     * di16[r, pl.ds(0, 16)] * 0.0625)

        pltpu.sync_copy(ob16, out.at[pl.ds(base, sb)])


def _final_call(u0, u1, u2, u3, dsq16, uq, iq):
    bq = 4096 // (NC * NS)
    kern = pl.kernel(
        _final_body,
        out_type=jax.ShapeDtypeStruct((4096, 16), _f32),
        mesh=_mesh(),
        compiler_params=pltpu.CompilerParams(use_tc_tiling_on_sc=False),
        scratch_types=[
            pltpu.VMEM((64,), _i32),
            pltpu.VMEM((64,), _i32),
            pltpu.VMEM((64, D), _f32),
            pltpu.VMEM((64, D), _f32),
            pltpu.VMEM((64, D), _f32),
            pltpu.VMEM((64, D), _f32),
            pltpu.VMEM((64, D), _f32),
            pltpu.VMEM((64, D), _f32),
            pltpu.VMEM((64, D), _f32),
            pltpu.VMEM((64, D), _f32),
            pltpu.VMEM((64, 16), _f32),
            pltpu.VMEM((64, 16), _f32),
            pltpu.VMEM((64, 16), _f32),
        ],
    )
    return kern(u0, u1, u2, u3, dsq16, uq, iq)


def kernel(users, items, user_emb, item_emb, edge_u, edge_i, graph_vals):
    eu = edge_u.astype(_i32)
    ei = edge_i.astype(_i32)
    us = users.astype(_i32)
    it = items.astype(_i32)
    pad = EP_SIDE - E
    zpad = jnp.zeros((pad,), _i32)
    npad = jnp.full((pad,), -1, _i32)

    src_u = jnp.concatenate([ei + SIDE_P, zpad])
    dst_u = jnp.concatenate([eu, npad])
    src_i = jnp.concatenate([eu, zpad])
    dst_i = jnp.concatenate([ei, npad])
    src2 = jnp.stack([src_u, src_i]).reshape(NC, NS, NCH, CK)
    dst2 = jnp.stack([dst_u, dst_i]).reshape(NC, NS, NCH, CK)

    rpad = jnp.zeros((SIDE_P - SIDE, D), _f32)
    all_emb_p = jnp.concatenate([user_emb, rpad, item_emb, rpad], axis=0)

    deg = _deg_call(dst2)[:, 0]
    dm = jnp.maximum(deg, 1.0)
    dinv = 1.0 / dm
    dsqi2d = lax.rsqrt(dm)[:, None]
    dsq16 = jnp.broadcast_to(jnp.sqrt(dm)[:, None], (NP, 16))

    u0 = _u0_call(all_emb_p, dsqi2d)
    u1 = _layer_call(u0, src2, dst2, dinv)
    u2 = _layer_call(u1, src2, dst2, dinv)
    u3 = _layer_call(u2, src2, dst2, dinv)

    return jnp.sum(_final_call(u0, u1, u2, u3, dsq16, us, it + SIDE_P), axis=1)
